# baseline (speedup 1.0000x reference)
import functools

import jax
import jax.numpy as jnp
from jax import lax
from jax.experimental import pallas as pl
from jax.experimental.pallas import tpu as pltpu
from jax.experimental.pallas import tpu_sc as plsc

_CHUNK = 128
_W = 128


@functools.lru_cache(maxsize=None)
def _build(B, D, VF):
    info = plsc.get_sparse_core_info()
    NC, NS = info.num_cores, info.num_subcores
    NW = NC * NS
    b_per_w = B // NW
    assert B % (NW * _CHUNK) == 0 and D % 16 == 0
    cpw = b_per_w // _CHUNK

    mesh = plsc.VectorSubcoreMesh(core_axis_name="c", subcore_axis_name="s")

    @functools.partial(
        pl.kernel,
        mesh=mesh,
        out_type=jax.ShapeDtypeStruct((2 * D, B), jnp.float32),
        compiler_params=pltpu.CompilerParams(needs_layout_passes=False),
        scratch_types=[
            pltpu.VMEM((cpw, _CHUNK), jnp.int32),
            pltpu.VMEM((cpw, _CHUNK), jnp.int32),
            pltpu.VMEM((cpw, _CHUNK), jnp.int32),
            pltpu.VMEM((cpw, _CHUNK), jnp.int32),
            pltpu.VMEM((_CHUNK, _W), jnp.float32),
            pltpu.VMEM((_CHUNK, _W), jnp.float32),
            pltpu.VMEM((_CHUNK, _W), jnp.float32),
            pltpu.VMEM((_CHUNK, _W), jnp.float32),
            pltpu.VMEM((2 * D, b_per_w), jnp.float32),
            pltpu.SemaphoreType.DMA,
        ],
    )
    def k(ufid_hbm, ucol_hbm, bfid_hbm, bcol_hbm, utab_hbm, btab_hbm, out_hbm,
          ufid_v, ucol_v, bfid_v, bcol_v, u_v0, u_v1, b_v0, b_v1, comb_v,
          gsem):
        u_ring, b_ring = (u_v0, u_v1), (b_v0, b_v1)
        wid = lax.axis_index("s") * NC + lax.axis_index("c")
        base = wid * b_per_w
        row0 = wid * cpw
        pltpu.sync_copy(ufid_hbm.at[pl.ds(row0, cpw)], ufid_v)
        pltpu.sync_copy(ucol_hbm.at[pl.ds(row0, cpw)], ucol_v)
        pltpu.sync_copy(bfid_hbm.at[pl.ds(row0, cpw)], bfid_v)
        pltpu.sync_copy(bcol_hbm.at[pl.ds(row0, cpw)], bcol_v)

        def fire(j):
            s = j % 2
            cu = pltpu.async_copy(utab_hbm.at[ufid_v.at[j]], u_ring[s], gsem)
            cb = pltpu.async_copy(btab_hbm.at[bfid_v.at[j]], b_ring[s], gsem)
            return cu, cb

        iota16 = lax.iota(jnp.int32, 16)

        def select_transpose(j):
            s = j % 2
            u_v, b_v = u_ring[s], b_ring[s]

            def gbody(g, _):
                rows = iota16 + g * 16
                pos = pl.ds(j * _CHUNK + g * 16, 16)
                qu = ucol_v[j, pl.ds(g * 16, 16)]
                qb = bcol_v[j, pl.ds(g * 16, 16)]
                for d in range(D):
                    comb_v[d, pos] = plsc.load_gather(u_v, [rows, qu + d])
                    comb_v[D + d, pos] = plsc.load_gather(b_v, [rows, qb + d])
                return 0
            lax.fori_loop(0, _CHUNK // 16, gbody, 0)

        pending = fire(0)
        for j in range(cpw):
            for c in pending:
                c.wait()
            if j + 1 < cpw:
                pending = fire(j + 1)
            select_transpose(j)
        pltpu.sync_copy(comb_v, out_hbm.at[:, pl.ds(base, b_per_w)])

    return k


def kernel(user_ids, book_ids, user_table, book_table):
    B = user_ids.shape[0]
    V, D = user_table.shape
    fold = _W // D
    pad_rows = (-V) % fold
    VF = (V + pad_rows) * D // _W

    uids = user_ids.astype(jnp.int32)
    bids = book_ids.astype(jnp.int32)
    ufid = (uids // fold).reshape(B // _CHUNK, _CHUNK)
    ucol = ((uids % fold) * D).reshape(B // _CHUNK, _CHUNK)
    bfid = (bids // fold).reshape(B // _CHUNK, _CHUNK)
    bcol = ((bids % fold) * D).reshape(B // _CHUNK, _CHUNK)
    zpad = jnp.zeros((pad_rows, D), jnp.float32)
    utabf = jnp.concatenate([user_table, zpad], axis=0).reshape(VF, _W)
    btabf = jnp.concatenate([book_table, zpad], axis=0).reshape(VF, _W)

    k = _build(B, D, VF)
    out_t = k(ufid, ucol, bfid, bcol, utabf, btabf)
    return out_t.T


# speedup vs baseline: 1.6527x; 1.6527x over previous
import functools

import jax
import jax.numpy as jnp
from jax import lax
from jax.experimental import pallas as pl
from jax.experimental.pallas import tpu as pltpu
from jax.experimental.pallas import tpu_sc as plsc

_CHUNK = 128
_W = 128


@functools.lru_cache(maxsize=None)
def _build(B, D):
    info = plsc.get_sparse_core_info()
    NC, NS = info.num_cores, info.num_subcores
    NW = NC * NS
    b_per_w = B // NW
    assert B % (NW * _CHUNK) == 0 and D % 16 == 0
    cpw = b_per_w // _CHUNK

    mesh = plsc.VectorSubcoreMesh(core_axis_name="c", subcore_axis_name="s")

    @functools.partial(
        pl.kernel,
        mesh=mesh,
        out_type=jax.ShapeDtypeStruct((B, 2 * D), jnp.float32),
        scratch_types=[
            pltpu.VMEM((_CHUNK * cpw,), jnp.int32),
            pltpu.VMEM((_CHUNK * cpw,), jnp.int32),
            pltpu.VMEM((_CHUNK, _W), jnp.float32),
            pltpu.VMEM((_CHUNK, _W), jnp.float32),
            pltpu.VMEM((_CHUNK, _W), jnp.float32),
            pltpu.VMEM((_CHUNK, _W), jnp.float32),
            pltpu.VMEM((_CHUNK, 2 * D), jnp.float32),
            pltpu.VMEM((_CHUNK, 2 * D), jnp.float32),
            pltpu.SemaphoreType.DMA,
            pltpu.SemaphoreType.DMA,
        ],
    )
    def k(uids_hbm, bids_hbm, utab_hbm, btab_hbm, out_hbm,
          uidx_v, bidx_v, u_v0, u_v1, b_v0, b_v1, comb_v0, comb_v1,
          gsem, osem):
        u_ring, b_ring = (u_v0, u_v1), (b_v0, b_v1)
        comb_ring = (comb_v0, comb_v1)
        wid = lax.axis_index("s") * NC + lax.axis_index("c")
        base = wid * b_per_w
        pltpu.sync_copy(uids_hbm.at[pl.ds(base, b_per_w)], uidx_v)
        pltpu.sync_copy(bids_hbm.at[pl.ds(base, b_per_w)], bidx_v)

        def fire(j):
            s = j % 2
            rows = pl.ds(j * _CHUNK, _CHUNK)
            cu = pltpu.async_copy(utab_hbm.at[uidx_v.at[rows]], u_ring[s], gsem)
            cb = pltpu.async_copy(btab_hbm.at[bidx_v.at[rows]], b_ring[s], gsem)
            return cu, cb

        def interleave(s):
            u_v, b_v, comb_v = u_ring[s], b_ring[s], comb_ring[s]

            def body(i, _):
                for c in range(D // 16):
                    comb_v[i, pl.ds(16 * c, 16)] = u_v[i, pl.ds(16 * c, 16)]
                    comb_v[i, pl.ds(D + 16 * c, 16)] = b_v[i, pl.ds(16 * c, 16)]
                return 0
            lax.fori_loop(0, _CHUNK, body, 0)

        pending = fire(0)
        out_cp = None
        for j in range(cpw):
            s = j % 2
            for c in pending:
                c.wait()
            if j + 1 < cpw:
                pending = fire(j + 1)
            interleave(s)
            if out_cp is not None:
                out_cp.wait()
            out_cp = pltpu.async_copy(
                comb_ring[s], out_hbm.at[pl.ds(base + j * _CHUNK, _CHUNK)],
                osem)
        out_cp.wait()

    return k


def kernel(user_ids, book_ids, user_table, book_table):
    B = user_ids.shape[0]
    V, D = user_table.shape
    uids = user_ids.astype(jnp.int32)
    bids = book_ids.astype(jnp.int32)
    utab = jnp.pad(user_table, ((0, 0), (0, _W - D)))
    btab = jnp.pad(book_table, ((0, 0), (0, _W - D)))
    k = _build(B, D)
    return k(uids, bids, utab, btab)
